# baseline (speedup 1.0000x reference)
import functools

import jax
import jax.numpy as jnp
from jax import lax
from jax.experimental import pallas as pl
from jax.experimental.pallas import tpu as pltpu
from jax.experimental.pallas import tpu_sc as plsc

_NC = 2
_NS = 16
_NW = _NC * _NS
_CHUNK = 128


def _gather_body(b_per_w, n_chunks, D,
                 uidx_hbm, iidx_hbm, uemb_hbm, iemb_hbm,
                 uout_hbm, iout_hbm,
                 uidx_v, iidx_v, urows_v, irows_v, sem):
    wid = lax.axis_index("s") * _NC + lax.axis_index("c")
    base = wid * b_per_w
    pltpu.sync_copy(uidx_hbm.at[wid], uidx_v)
    pltpu.sync_copy(iidx_hbm.at[wid], iidx_v)
    copies = []
    for c in range(n_chunks):
        rows = pl.ds(c * _CHUNK, _CHUNK)
        copies.append(pltpu.make_async_copy(
            uemb_hbm.at[uidx_v.at[c]], urows_v.at[rows], sem))
        copies.append(pltpu.make_async_copy(
            iemb_hbm.at[iidx_v.at[c]], irows_v.at[rows], sem))
    for cp in copies:
        cp.start()
    for cp in copies:
        cp.wait()
    pltpu.sync_copy(urows_v, uout_hbm.at[pl.ds(base, b_per_w)])
    pltpu.sync_copy(irows_v, iout_hbm.at[pl.ds(base, b_per_w)])


@functools.partial(jax.jit, static_argnums=(4, 5))
def _sc_gather(uidx, iidx, uemb, iemb, B, D):
    b_per_w = B // _NW
    n_chunks = b_per_w // _CHUNK
    mesh = plsc.VectorSubcoreMesh(core_axis_name="c", subcore_axis_name="s")
    body = functools.partial(_gather_body, b_per_w, n_chunks, D)
    kern = pl.kernel(
        body,
        out_type=[
            jax.ShapeDtypeStruct((B, D), jnp.float32),
            jax.ShapeDtypeStruct((B, D), jnp.float32),
        ],
        mesh=mesh,
        scratch_types=[
            pltpu.VMEM((n_chunks, _CHUNK), jnp.int32),
            pltpu.VMEM((n_chunks, _CHUNK), jnp.int32),
            pltpu.VMEM((b_per_w, D), jnp.float32),
            pltpu.VMEM((b_per_w, D), jnp.float32),
            pltpu.SemaphoreType.DMA,
        ],
        compiler_params=pltpu.CompilerParams(use_tc_tiling_on_sc=False),
    )
    uidx3 = uidx.reshape(_NW, n_chunks, _CHUNK)
    iidx3 = iidx.reshape(_NW, n_chunks, _CHUNK)
    return kern(uidx3, iidx3, uemb, iemb)


def _mlp_body(u_ref, i_ref, w1a_ref, w1b_ref, b1_ref, w2_ref, b2_ref,
              wp_ref, bp_ref, o_ref):
    u = u_ref[...]
    it = i_ref[...]
    h1 = jnp.dot(u, w1a_ref[...], preferred_element_type=jnp.float32)
    h1 += jnp.dot(it, w1b_ref[...], preferred_element_type=jnp.float32)
    h1 = jnp.maximum(h1 + b1_ref[...], 0.0)
    h2 = jnp.dot(h1, w2_ref[...], preferred_element_type=jnp.float32)
    h2 = jnp.maximum(h2 + b2_ref[...], 0.0)
    p = jnp.dot(h2, wp_ref[...], preferred_element_type=jnp.float32)
    o_ref[...] = jax.nn.sigmoid(p + bp_ref[...])


def _tc_mlp(u, it, W1, b1, W2, b2, Wp, bp, B, BK):
    D = u.shape[1]
    w1a = W1[:, :D].T
    w1b = W1[:, D:].T
    w2 = W2.T
    wp = Wp.T
    b1r = b1.reshape(1, -1)
    b2r = b2.reshape(1, -1)
    bpr = bp.reshape(1, -1)
    grid = B // BK

    def full(shape):
        return pl.BlockSpec(shape, lambda i: (0,) * len(shape))

    out = pl.pallas_call(
        _mlp_body,
        grid=(grid,),
        in_specs=[
            pl.BlockSpec((BK, D), lambda i: (i, 0)),
            pl.BlockSpec((BK, D), lambda i: (i, 0)),
            full(w1a.shape), full(w1b.shape), full(b1r.shape),
            full(w2.shape), full(b2r.shape),
            full(wp.shape), full(bpr.shape),
        ],
        out_specs=pl.BlockSpec((BK, 1), lambda i: (i, 0)),
        out_shape=jax.ShapeDtypeStruct((B, 1), jnp.float32),
    )(u, it, w1a, w1b, b1r, w2, b2r, wp, bpr)
    return out


def kernel(user_indices, item_indices, user_emb, item_emb,
           W1, b1, W2, b2, Wp, bp):
    B = user_indices.shape[0]
    D = user_emb.shape[1]
    u_rows, i_rows = _sc_gather(
        user_indices.astype(jnp.int32), item_indices.astype(jnp.int32),
        user_emb, item_emb, B, D)
    out = _tc_mlp(u_rows, i_rows, W1, b1, W2, b2, Wp, bp, B, 2048)
    return jnp.squeeze(out, axis=-1)
